# baseline (speedup 1.0000x reference)
import functools
import jax
import jax.numpy as jnp
from jax import lax
from jax.experimental import pallas as pl
from jax.experimental.pallas import tpu as pltpu
from jax.experimental.pallas import tpu_sc as plsc

_NBUF = 4


def kernel(x, pos_emb):
    seq_len = x.shape[1]
    d_model = pos_emb.shape[1]
    info = plsc.get_sparse_core_info()
    nw = info.num_cores * info.num_subcores
    rows_per_w = seq_len // nw
    chunk = 32
    nchunks = rows_per_w // chunk
    mesh = plsc.VectorSubcoreMesh(core_axis_name="c", subcore_axis_name="s")

    @functools.partial(
        pl.kernel,
        mesh=mesh,
        out_type=jax.ShapeDtypeStruct((seq_len, d_model), jnp.float32),
        scratch_types=[
            pltpu.VMEM((_NBUF, chunk, d_model), jnp.float32),
            pltpu.SemaphoreType.DMA((_NBUF,)),
            pltpu.SemaphoreType.DMA((_NBUF,)),
        ],
    )
    def sc_copy(table_hbm, out_hbm, buf, in_sems, out_sems):
        wid = lax.axis_index("s") * info.num_cores + lax.axis_index("c")
        base = wid * rows_per_w

        def in_copy(j):
            return pltpu.make_async_copy(
                table_hbm.at[pl.ds(base + j * chunk, chunk), :],
                buf.at[j % _NBUF],
                in_sems.at[j % _NBUF],
            )

        def out_copy(j):
            return pltpu.make_async_copy(
                buf.at[j % _NBUF],
                out_hbm.at[pl.ds(base + j * chunk, chunk), :],
                out_sems.at[j % _NBUF],
            )

        for j in range(_NBUF):
            in_copy(j).start()
        for j in range(nchunks):
            in_copy(j).wait()
            out_copy(j).start()
            if j + _NBUF < nchunks:
                out_copy(j).wait()
                in_copy(j + _NBUF).start()
        for j in range(nchunks - _NBUF, nchunks):
            out_copy(j).wait()

    return sc_copy(pos_emb)


# speedup vs baseline: 2.0891x; 2.0891x over previous
import jax
import jax.numpy as jnp
from jax.experimental import pallas as pl
from jax.experimental.pallas import tpu as pltpu


def _copy_block(in_ref, out_ref):
    out_ref[...] = in_ref[...]


def kernel(x, pos_emb):
    seq_len = x.shape[1]
    d_model = pos_emb.shape[1]
    block_rows = 3072
    num_blocks = pl.cdiv(seq_len, block_rows)
    return pl.pallas_call(
        _copy_block,
        grid=(num_blocks,),
        in_specs=[pl.BlockSpec((block_rows, d_model), lambda i: (i, 0))],
        out_specs=pl.BlockSpec((block_rows, d_model), lambda i: (i, 0)),
        out_shape=jax.ShapeDtypeStruct((seq_len, d_model), pos_emb.dtype),
    )(pos_emb)
